# baseline (speedup 1.0000x reference)
import functools

import jax
import jax.numpy as jnp
from jax import lax
from jax.experimental import pallas as pl
from jax.experimental.pallas import tpu as pltpu
from jax.experimental.pallas import tpu_sc as plsc

_NUM_CORES = 2
_NUM_SUBCORES = 16
_NW = _NUM_CORES * _NUM_SUBCORES

_D = 64
_L = 128
_NBUF = 4


def _sc_gather_t(table, idx_flat, T, S):
    mesh = plsc.VectorSubcoreMesh(core_axis_name="c", subcore_axis_name="s")
    n_units = (T * S) // _L
    upw = n_units // _NW
    n_outer = upw // _NBUF
    sb_per_t = S // _L

    @functools.partial(
        pl.kernel,
        out_type=jax.ShapeDtypeStruct((T, _D // 8, S // _L, 8, _L),
                                      jnp.float32),
        mesh=mesh,
        scratch_types=(
            [pltpu.VMEM((_L,), jnp.int32) for _ in range(_NBUF)]
            + [pltpu.VMEM((_L, _D), jnp.float32) for _ in range(_NBUF)]
            + [pltpu.VMEM((_D, _L + 1), jnp.float32)
               for _ in range(_NBUF)]
            + [pltpu.SemaphoreType.DMA for _ in range(3 * _NBUF)]
        ),
        compiler_params=pltpu.CompilerParams(use_tc_tiling_on_sc=False, needs_layout_passes=False),
    )
    def k(table_hbm, idx_hbm, out_hbm, *bufs):
        idx_v = bufs[:_NBUF]
        rows_v = bufs[_NBUF:2 * _NBUF]
        tr_v = bufs[2 * _NBUF:3 * _NBUF]
        isem = bufs[3 * _NBUF:4 * _NBUF]
        gsem = bufs[4 * _NBUF:5 * _NBUF]
        wsem = bufs[5 * _NBUF:]
        wid = lax.axis_index("s") * _NUM_CORES + lax.axis_index("c")
        base = wid * upw

        i16 = lax.iota(jnp.int32, 16)
        d_idx = [i16 + (16 * j2) for j2 in range(4)]

        def fire_gather(b, u):
            pltpu.async_copy(table_hbm.at[idx_v[b]], rows_v[b], gsem[b])

        def wait_gather(b):
            pltpu.make_async_copy(table_hbm.at[idx_v[b]], rows_v[b],
                                  gsem[b]).wait()

        def fire_idx(b, u):
            pltpu.async_copy(idx_hbm.at[pl.ds(u * _L, _L)], idx_v[b],
                             isem[b])

        def wait_idx(b, u):
            pltpu.make_async_copy(idx_hbm.at[pl.ds(u * _L, _L)], idx_v[b],
                                  isem[b]).wait()

        def fire_writes(b, u):
            t = lax.shift_right_logical(u, 7)
            sb = lax.bitwise_and(u, sb_per_t - 1)
            for dt in range(_D // 8):
                pltpu.async_copy(
                    tr_v[b].at[pl.ds(dt * 8, 8), pl.ds(0, _L)],
                    out_hbm.at[t, dt, sb], wsem[b])

        def wait_writes(b, u):
            t = lax.shift_right_logical(u, 7)
            sb = lax.bitwise_and(u, sb_per_t - 1)
            for dt in range(_D // 8):
                pltpu.make_async_copy(
                    tr_v[b].at[pl.ds(dt * 8, 8), pl.ds(0, _L)],
                    out_hbm.at[t, dt, sb], wsem[b]).wait()

        def transpose(b):
            zero16 = jnp.zeros((16,), jnp.int32)
            for s in range(_L):
                ssp = zero16 + s
                for j2 in range(4):
                    val = rows_v[b][s, pl.ds(16 * j2, 16)]
                    plsc.store_scatter(tr_v[b], [d_idx[j2], ssp], val)

        for b in range(_NBUF):
            fire_idx(b, base + b)
            wait_idx(b, base + b)
            fire_gather(b, base + b)

        def outer(ti, carry):
            for b in range(_NBUF):
                u = base + ti * _NBUF + b
                wait_gather(b)

                @pl.when(ti < n_outer - 1)
                def _():
                    fire_idx(b, u + _NBUF)

                @pl.when(ti > 0)
                def _():
                    wait_writes(b, u - _NBUF)

                transpose(b)
                fire_writes(b, u)

                @pl.when(ti < n_outer - 1)
                def _():
                    wait_idx(b, u + _NBUF)
                    fire_gather(b, u + _NBUF)
            return carry

        lax.fori_loop(0, n_outer, outer, 0)
        for b in range(_NBUF):
            wait_writes(b, base + (n_outer - 1) * _NBUF + b)

    return k(table, idx_flat)


def kernel(token_ids, weight):
    S, T = token_ids.shape
    D = weight.shape[1]
    idx_t = token_ids.T.reshape(S * T).astype(jnp.int32)
    out5 = _sc_gather_t(weight, idx_t, T, S)
    return out5.transpose(2, 4, 0, 1, 3).reshape(S, T, D)


# speedup vs baseline: 1.1989x; 1.1989x over previous
import functools

import jax
import jax.numpy as jnp
from jax import lax
from jax.experimental import pallas as pl
from jax.experimental.pallas import tpu as pltpu
from jax.experimental.pallas import tpu_sc as plsc

_NUM_CORES = 2
_NUM_SUBCORES = 16
_NW = _NUM_CORES * _NUM_SUBCORES

_D = 64
_L = 128
_NBUF = 4


def _sc_gather_t(table, idx_flat, T, S):
    mesh = plsc.VectorSubcoreMesh(core_axis_name="c", subcore_axis_name="s")
    n_units = (T * S) // _L
    upw = n_units // _NW
    n_outer = upw // _NBUF
    sb_per_t = S // _L

    @functools.partial(
        pl.kernel,
        out_type=jax.ShapeDtypeStruct((T, _D // 8, S // _L, 8, _L),
                                      jnp.float32),
        mesh=mesh,
        scratch_types=(
            [pltpu.VMEM((_L,), jnp.int32) for _ in range(_NBUF)]
            + [pltpu.VMEM((_L, _D), jnp.float32) for _ in range(_NBUF)]
            + [pltpu.VMEM((_D, _L + 1), jnp.float32)
               for _ in range(_NBUF)]
            + [pltpu.SemaphoreType.DMA for _ in range(3 * _NBUF)]
        ),
        compiler_params=pltpu.CompilerParams(use_tc_tiling_on_sc=False, needs_layout_passes=False),
    )
    def k(table_hbm, idx_hbm, out_hbm, *bufs):
        idx_v = bufs[:_NBUF]
        rows_v = bufs[_NBUF:2 * _NBUF]
        tr_v = bufs[2 * _NBUF:3 * _NBUF]
        isem = bufs[3 * _NBUF:4 * _NBUF]
        gsem = bufs[4 * _NBUF:5 * _NBUF]
        wsem = bufs[5 * _NBUF:]
        wid = lax.axis_index("s") * _NUM_CORES + lax.axis_index("c")
        base = wid * upw

        i16 = lax.iota(jnp.int32, 16)
        d_idx = [i16 + (16 * j2) for j2 in range(4)]

        def fire_gather(b, u):
            pltpu.async_copy(table_hbm.at[idx_v[b]], rows_v[b], gsem[b])

        def wait_gather(b):
            pltpu.make_async_copy(table_hbm.at[idx_v[b]], rows_v[b],
                                  gsem[b]).wait()

        def fire_idx(b, u):
            pltpu.async_copy(idx_hbm.at[pl.ds(u * _L, _L)], idx_v[b],
                             isem[b])

        def wait_idx(b, u):
            pltpu.make_async_copy(idx_hbm.at[pl.ds(u * _L, _L)], idx_v[b],
                                  isem[b]).wait()

        def fire_writes(b, u):
            t = lax.shift_right_logical(u, 7)
            sb = lax.bitwise_and(u, sb_per_t - 1)
            for dt in range(_D // 8):
                pltpu.async_copy(
                    tr_v[b].at[pl.ds(dt * 8, 8), pl.ds(0, _L)],
                    out_hbm.at[t, dt, sb], wsem[b])

        def wait_writes(b, u):
            t = lax.shift_right_logical(u, 7)
            sb = lax.bitwise_and(u, sb_per_t - 1)
            for dt in range(_D // 8):
                pltpu.make_async_copy(
                    tr_v[b].at[pl.ds(dt * 8, 8), pl.ds(0, _L)],
                    out_hbm.at[t, dt, sb], wsem[b]).wait()

        def transpose(b):
            def tbody(g, ssp):
                for q in range(8):
                    s = g * 8 + q
                    sq = ssp + q
                    for j2 in range(4):
                        val = rows_v[b][s, pl.ds(16 * j2, 16)]
                        plsc.store_scatter(tr_v[b], [d_idx[j2], sq], val)
                return ssp + 8

            lax.fori_loop(0, _L // 8, tbody, jnp.zeros((16,), jnp.int32))

        for b in range(_NBUF):
            fire_idx(b, base + b)
            wait_idx(b, base + b)
            fire_gather(b, base + b)

        def outer(ti, carry):
            for b in range(_NBUF):
                u = base + ti * _NBUF + b
                wait_gather(b)

                @pl.when(ti < n_outer - 1)
                def _():
                    fire_idx(b, u + _NBUF)

                @pl.when(ti > 0)
                def _():
                    wait_writes(b, u - _NBUF)

                transpose(b)
                fire_writes(b, u)

                @pl.when(ti < n_outer - 1)
                def _():
                    wait_idx(b, u + _NBUF)
                    fire_gather(b, u + _NBUF)
            return carry

        lax.fori_loop(0, n_outer, outer, 0)
        for b in range(_NBUF):
            wait_writes(b, base + (n_outer - 1) * _NBUF + b)

    return k(table, idx_flat)


def kernel(token_ids, weight):
    S, T = token_ids.shape
    D = weight.shape[1]
    idx_t = token_ids.T.reshape(S * T).astype(jnp.int32)
    out5 = _sc_gather_t(weight, idx_t, T, S)
    return out5.transpose(2, 4, 0, 1, 3).reshape(S, T, D)
